# baseline (speedup 1.0000x reference)
import jax
import jax.numpy as jnp
from jax.experimental import pallas as pl
from jax.experimental.pallas import tpu as pltpu

N_USERS = 8192
N_ITEMS = 2048
BETA = 0.3
EPS = 1e-12

_BU = 1024
_NB = N_USERS // _BU
_T = 512
_NT = N_ITEMS // _T


def _fused_body(x_ref, g_ref, ni_ref):
    k = pl.program_id(0)

    @pl.when(k == 0)
    def _():
        ni_ref[...] = jnp.zeros((1, N_ITEMS), jnp.float32)
        for i in range(_NT):
            g_ref[pl.ds(i * _T, _T), pl.ds(i * _T, (_NT - i) * _T)] = (
                jnp.zeros((_T, (_NT - i) * _T), jnp.float32))

    x = x_ref[...]
    xb = x.astype(jnp.bfloat16)
    n_u = jnp.sum(x, axis=1, keepdims=True)
    ones_l = jnp.ones((_BU, 8), jnp.bfloat16)
    col = jax.lax.dot_general(
        ones_l, xb, dimension_numbers=(((0,), (0,)), ((), ())),
        preferred_element_type=jnp.float32)[0:1]
    sw = jnp.sqrt(jnp.power(n_u + EPS, -BETA)).astype(jnp.bfloat16)
    y = sw * xb
    ni_ref[...] += col

    for i in range(_NT):
        yi = y[:, i * _T:(i + 1) * _T]
        for j in range(i, _NT):
            yj = y[:, j * _T:(j + 1) * _T]
            blk = jax.lax.dot_general(
                yi, yj,
                dimension_numbers=(((0,), (0,)), ((), ())),
                preferred_element_type=jnp.float32)
            g_ref[pl.ds(i * _T, _T), pl.ds(j * _T, _T)] += blk


_N_PAIRS = (_NT * (_NT - 1)) // 2


def _pair_i(t):
    return (t >= 3).astype(jnp.int32) + (t >= 5).astype(jnp.int32)


def _pair_j(t):
    return t + 1 - 2 * (t >= 3).astype(jnp.int32) - (t >= 5).astype(jnp.int32)


def _mirror_body(gu_ref, gl_ref):
    gl_ref[...] = gu_ref[...].T


def kernel(X):
    G, ni = pl.pallas_call(
        _fused_body,
        grid=(_NB,),
        in_specs=[pl.BlockSpec((_BU, N_ITEMS), lambda k: (k, 0))],
        out_specs=[
            pl.BlockSpec((N_ITEMS, N_ITEMS), lambda k: (0, 0)),
            pl.BlockSpec((1, N_ITEMS), lambda k: (0, 0)),
        ],
        out_shape=[
            jax.ShapeDtypeStruct((N_ITEMS, N_ITEMS), jnp.float32),
            jax.ShapeDtypeStruct((1, N_ITEMS), jnp.float32),
        ],
    )(X)

    G = pl.pallas_call(
        _mirror_body,
        grid=(_N_PAIRS,),
        in_specs=[pl.BlockSpec((_T, _T), lambda t: (_pair_i(t), _pair_j(t)))],
        out_specs=pl.BlockSpec((_T, _T), lambda t: (_pair_j(t), _pair_i(t))),
        out_shape=jax.ShapeDtypeStruct((N_ITEMS, N_ITEMS), jnp.float32),
        input_output_aliases={0: 0},
    )(G)
    return (G, ni.reshape(N_ITEMS))


# speedup vs baseline: 1.0992x; 1.0992x over previous
import jax
import jax.numpy as jnp
from jax.experimental import pallas as pl
from jax.experimental.pallas import tpu as pltpu

N_USERS = 8192
N_ITEMS = 2048
BETA = 0.3
EPS = 1e-12

_BU = 1024
_NB = N_USERS // _BU
_T = 512
_NT = N_ITEMS // _T


def _fused_body(x_ref, g_ref, ni_ref):
    k = pl.program_id(0)

    @pl.when(k == 0)
    def _():
        ni_ref[...] = jnp.zeros((1, N_ITEMS), jnp.float32)
        for i in range(_NT):
            g_ref[pl.ds(i * _T, _T), pl.ds(i * _T, (_NT - i) * _T)] = (
                jnp.zeros((_T, (_NT - i) * _T), jnp.float32))

    x = x_ref[...]
    xb = x.astype(jnp.bfloat16)
    n_u = jnp.sum(x, axis=1, keepdims=True)
    ones_l = jnp.ones((_BU, 8), jnp.bfloat16)
    col = jax.lax.dot_general(
        ones_l, xb, dimension_numbers=(((0,), (0,)), ((), ())),
        preferred_element_type=jnp.float32)[0:1]
    sw = jnp.sqrt(jnp.power(n_u + EPS, -BETA)).astype(jnp.bfloat16)
    y = sw * xb
    ni_ref[...] += col

    for i in range(_NT):
        yi = y[:, i * _T:(i + 1) * _T]
        for j in range(i, _NT):
            yj = y[:, j * _T:(j + 1) * _T]
            blk = jax.lax.dot_general(
                yi, yj,
                dimension_numbers=(((0,), (0,)), ((), ())),
                preferred_element_type=jnp.float32)
            g_ref[pl.ds(i * _T, _T), pl.ds(j * _T, _T)] += blk

    @pl.when(k == _NB - 1)
    def _():
        for i in range(_NT):
            for j in range(i + 1, _NT):
                g_ref[pl.ds(j * _T, _T), pl.ds(i * _T, _T)] = (
                    g_ref[pl.ds(i * _T, _T), pl.ds(j * _T, _T)].T)


def kernel(X):
    G, ni = pl.pallas_call(
        _fused_body,
        grid=(_NB,),
        in_specs=[pl.BlockSpec((_BU, N_ITEMS), lambda k: (k, 0))],
        out_specs=[
            pl.BlockSpec((N_ITEMS, N_ITEMS), lambda k: (0, 0)),
            pl.BlockSpec((1, N_ITEMS), lambda k: (0, 0)),
        ],
        out_shape=[
            jax.ShapeDtypeStruct((N_ITEMS, N_ITEMS), jnp.float32),
            jax.ShapeDtypeStruct((1, N_ITEMS), jnp.float32),
        ],
    )(X)
    return (G, ni.reshape(N_ITEMS))


# speedup vs baseline: 1.1613x; 1.0565x over previous
import jax
import jax.numpy as jnp
from jax.experimental import pallas as pl
from jax.experimental.pallas import tpu as pltpu

N_USERS = 8192
N_ITEMS = 2048
BETA = 0.3
EPS = 1e-12

_BU = 1024
_NB = N_USERS // _BU
_T = 256
_NT = N_ITEMS // _T


def _fused_body(x_ref, g_ref, ni_ref):
    k = pl.program_id(0)

    @pl.when(k == 0)
    def _():
        ni_ref[...] = jnp.zeros((1, N_ITEMS), jnp.float32)
        for i in range(_NT):
            g_ref[pl.ds(i * _T, _T), pl.ds(i * _T, (_NT - i) * _T)] = (
                jnp.zeros((_T, (_NT - i) * _T), jnp.float32))

    x = x_ref[...]
    xb = x.astype(jnp.bfloat16)
    n_u = jnp.sum(x, axis=1, keepdims=True)
    ones_l = jnp.ones((_BU, 8), jnp.bfloat16)
    col = jax.lax.dot_general(
        ones_l, xb, dimension_numbers=(((0,), (0,)), ((), ())),
        preferred_element_type=jnp.float32)[0:1]
    sw = jnp.sqrt(jnp.power(n_u + EPS, -BETA)).astype(jnp.bfloat16)
    y = sw * xb
    ni_ref[...] += col

    for i in range(_NT):
        yi = y[:, i * _T:(i + 1) * _T]
        for j in range(i, _NT):
            yj = y[:, j * _T:(j + 1) * _T]
            blk = jax.lax.dot_general(
                yi, yj,
                dimension_numbers=(((0,), (0,)), ((), ())),
                preferred_element_type=jnp.float32)
            g_ref[pl.ds(i * _T, _T), pl.ds(j * _T, _T)] += blk

    @pl.when(k == _NB - 1)
    def _():
        for i in range(_NT):
            for j in range(i + 1, _NT):
                g_ref[pl.ds(j * _T, _T), pl.ds(i * _T, _T)] = (
                    g_ref[pl.ds(i * _T, _T), pl.ds(j * _T, _T)].T)


def kernel(X):
    G, ni = pl.pallas_call(
        _fused_body,
        grid=(_NB,),
        in_specs=[pl.BlockSpec((_BU, N_ITEMS), lambda k: (k, 0))],
        out_specs=[
            pl.BlockSpec((N_ITEMS, N_ITEMS), lambda k: (0, 0)),
            pl.BlockSpec((1, N_ITEMS), lambda k: (0, 0)),
        ],
        out_shape=[
            jax.ShapeDtypeStruct((N_ITEMS, N_ITEMS), jnp.float32),
            jax.ShapeDtypeStruct((1, N_ITEMS), jnp.float32),
        ],
    )(X)
    return (G, ni.reshape(N_ITEMS))
